# baseline (speedup 1.0000x reference)
import functools

import jax
import jax.numpy as jnp
from jax import lax
from jax.experimental import pallas as pl
from jax.experimental.pallas import tpu as pltpu
from jax.experimental.pallas import tpu_sc as plsc

VOCAB = 1000000
SEQ = 200
DIM = 64
BATCH = 1024

NC = 2
NS = 16
NW = NC * NS
ROWS = BATCH * SEQ
RPW = ROWS // NW
CHUNK = SEQ
NCHUNK = RPW // CHUNK

_mesh = plsc.VectorSubcoreMesh(core_axis_name="c", subcore_axis_name="s")


@functools.partial(
    pl.kernel,
    out_type=jax.ShapeDtypeStruct((ROWS, DIM), jnp.float32),
    mesh=_mesh,
    compiler_params=pltpu.CompilerParams(use_tc_tiling_on_sc=False),
    scratch_types=[
        pltpu.VMEM((CHUNK,), jnp.int32),
        pltpu.VMEM((CHUNK, DIM), jnp.float32),
        pltpu.VMEM((SEQ, DIM), jnp.float32),
        pltpu.SemaphoreType.DMA,
    ],
)
def _embed(tok_hbm, idx_hbm, pos_hbm, out_hbm, idx_v, rows_v, pos_v, sem):
    wid = lax.axis_index("s") * NC + lax.axis_index("c")
    base = wid * RPW
    pltpu.sync_copy(pos_hbm, pos_v)

    def chunk_body(ci, _):
        off = base + ci * CHUNK
        pltpu.sync_copy(idx_hbm.at[pl.ds(off, CHUNK)], idx_v)
        pltpu.async_copy(tok_hbm.at[idx_v], rows_v, sem).wait()

        def row_body(r, _):
            for c in range(DIM // 16):
                sl = pl.ds(c * 16, 16)
                rows_v[r, sl] = rows_v[r, sl] + pos_v[r, sl]
            return 0

        lax.fori_loop(0, CHUNK, row_body, 0)
        pltpu.sync_copy(rows_v, out_hbm.at[pl.ds(off, CHUNK)])
        return 0

    lax.fori_loop(0, NCHUNK, chunk_body, 0)


def kernel(x, token_table, pos_table):
    xf = x.reshape(-1).astype(jnp.int32)
    out = _embed(token_table, xf, pos_table)
    return out.reshape(BATCH, SEQ, DIM)


# speedup vs baseline: 1.0699x; 1.0699x over previous
import functools

import jax
import jax.numpy as jnp
from jax import lax
from jax.experimental import pallas as pl
from jax.experimental.pallas import tpu as pltpu
from jax.experimental.pallas import tpu_sc as plsc

VOCAB = 1000000
SEQ = 200
DIM = 64
BATCH = 1024

NC = 2
NS = 16
NW = NC * NS
ROWS = BATCH * SEQ
RPW = ROWS // NW
CHUNK = SEQ
NCHUNK = RPW // CHUNK

_mesh = plsc.VectorSubcoreMesh(core_axis_name="c", subcore_axis_name="s")


@functools.partial(
    pl.kernel,
    out_type=jax.ShapeDtypeStruct((ROWS, DIM), jnp.float32),
    mesh=_mesh,
    compiler_params=pltpu.CompilerParams(use_tc_tiling_on_sc=False),
    scratch_types=[
        pltpu.VMEM((RPW,), jnp.int32),
        pltpu.VMEM((CHUNK, DIM), jnp.float32),
        pltpu.VMEM((CHUNK, DIM), jnp.float32),
        pltpu.VMEM((SEQ, DIM), jnp.float32),
        pltpu.SemaphoreType.DMA,
        pltpu.SemaphoreType.DMA,
        pltpu.SemaphoreType.DMA,
        pltpu.SemaphoreType.DMA,
    ],
)
def _embed(tok_hbm, idx_hbm, pos_hbm, out_hbm,
           idx_v, rows0, rows1, pos_v, g0, g1, s0, s1):
    wid = lax.axis_index("s") * NC + lax.axis_index("c")
    base = wid * RPW
    pltpu.sync_copy(idx_hbm.at[pl.ds(base, RPW)], idx_v)
    pltpu.sync_copy(pos_hbm, pos_v)

    def fill(rows):
        def body(r, _):
            for c in range(DIM // 16):
                sl = pl.ds(c * 16, 16)
                rows[r, sl] = pos_v[r, sl]
            return 0
        lax.fori_loop(0, CHUNK, body, 0)

    def start_gather(ci, rows, sem):
        pltpu.async_copy(
            tok_hbm.at[idx_v.at[pl.ds(ci * CHUNK, CHUNK)]], rows, sem,
            add=True)

    def wait_gather(rows, sem):
        pltpu.make_async_copy(
            tok_hbm.at[idx_v.at[pl.ds(0, CHUNK)]], rows, sem).wait()

    def start_store(ci, rows, sem):
        pltpu.async_copy(rows, out_hbm.at[pl.ds(base + ci * CHUNK, CHUNK)],
                         sem)

    def wait_store(rows, sem):
        pltpu.make_async_copy(rows, out_hbm.at[pl.ds(base, CHUNK)], sem).wait()

    def pair(g, _):
        ci0 = 2 * g
        ci1 = ci0 + 1

        @pl.when(g > 0)
        def _():
            wait_store(rows0, s0)

        fill(rows0)
        start_gather(ci0, rows0, g0)

        @pl.when(g > 0)
        def _():
            wait_store(rows1, s1)

        fill(rows1)
        start_gather(ci1, rows1, g1)

        wait_gather(rows0, g0)
        start_store(ci0, rows0, s0)
        wait_gather(rows1, g1)
        start_store(ci1, rows1, s1)
        return 0

    lax.fori_loop(0, NCHUNK // 2, pair, 0)
    wait_store(rows0, s0)
    wait_store(rows1, s1)


def kernel(x, token_table, pos_table):
    xf = x.reshape(-1).astype(jnp.int32)
    out = _embed(token_table, xf, pos_table)
    return out.reshape(BATCH, SEQ, DIM)
